# baseline (speedup 1.0000x reference)
import jax
import jax.numpy as jnp
from jax.experimental import pallas as pl
from jax.experimental.pallas import tpu as pltpu

_ROWS_PER_BLOCK = 512


def _copy_kernel(x_ref, o_ref):
    o_ref[...] = x_ref[...]


def kernel(x, embed):
    del embed
    rows, cols = x.shape
    grid = (rows // _ROWS_PER_BLOCK,)
    return pl.pallas_call(
        _copy_kernel,
        out_shape=jax.ShapeDtypeStruct(x.shape, x.dtype),
        grid=grid,
        in_specs=[pl.BlockSpec((_ROWS_PER_BLOCK, cols), lambda i: (i, 0))],
        out_specs=pl.BlockSpec((_ROWS_PER_BLOCK, cols), lambda i: (i, 0)),
    )(x)


# speedup vs baseline: 1.5895x; 1.5895x over previous
import jax
import jax.numpy as jnp
from jax.experimental import pallas as pl


def _copy_kernel(x_ref, o_ref):
    o_ref[...] = x_ref[...]


def kernel(x, embed):
    del embed
    return pl.pallas_call(
        _copy_kernel,
        out_shape=jax.ShapeDtypeStruct(x.shape, x.dtype),
    )(x)


# speedup vs baseline: 1.8945x; 1.1919x over previous
import jax
import jax.numpy as jnp
from jax.experimental import pallas as pl
from jax.experimental.pallas import tpu as pltpu

_HALF = 2048


def _copy_kernel(x_ref, o_ref, buf, sem_in, sem_out):
    in0 = pltpu.make_async_copy(x_ref.at[pl.ds(0, _HALF)], buf.at[0], sem_in.at[0])
    in1 = pltpu.make_async_copy(x_ref.at[pl.ds(_HALF, _HALF)], buf.at[1], sem_in.at[1])
    in0.start()
    in1.start()
    in0.wait()
    out0 = pltpu.make_async_copy(buf.at[0], o_ref.at[pl.ds(0, _HALF)], sem_out.at[0])
    out0.start()
    in1.wait()
    out1 = pltpu.make_async_copy(buf.at[1], o_ref.at[pl.ds(_HALF, _HALF)], sem_out.at[1])
    out1.start()
    out0.wait()
    out1.wait()


def kernel(x, embed):
    del embed
    rows, cols = x.shape
    return pl.pallas_call(
        _copy_kernel,
        out_shape=jax.ShapeDtypeStruct(x.shape, x.dtype),
        in_specs=[pl.BlockSpec(memory_space=pl.ANY)],
        out_specs=pl.BlockSpec(memory_space=pl.ANY),
        scratch_shapes=[
            pltpu.VMEM((2, _HALF, cols), x.dtype),
            pltpu.SemaphoreType.DMA((2,)),
            pltpu.SemaphoreType.DMA((2,)),
        ],
    )(x)


# speedup vs baseline: 2.0036x; 1.0576x over previous
import jax
import jax.numpy as jnp
from jax.experimental import pallas as pl
from jax.experimental.pallas import tpu as pltpu

_NCHUNK = 4
_ROWS = 4096 // _NCHUNK


def _copy_kernel(x_ref, o_ref, buf, sem_in, sem_out):
    ins = [
        pltpu.make_async_copy(
            x_ref.at[pl.ds(i * _ROWS, _ROWS)], buf.at[i], sem_in.at[i]
        )
        for i in range(_NCHUNK)
    ]
    outs = [
        pltpu.make_async_copy(
            buf.at[i], o_ref.at[pl.ds(i * _ROWS, _ROWS)], sem_out.at[i]
        )
        for i in range(_NCHUNK)
    ]
    for c in ins:
        c.start()
    for i in range(_NCHUNK):
        ins[i].wait()
        outs[i].start()
    for c in outs:
        c.wait()


def kernel(x, embed):
    del embed
    rows, cols = x.shape
    return pl.pallas_call(
        _copy_kernel,
        out_shape=jax.ShapeDtypeStruct(x.shape, x.dtype),
        in_specs=[pl.BlockSpec(memory_space=pl.ANY)],
        out_specs=pl.BlockSpec(memory_space=pl.ANY),
        scratch_shapes=[
            pltpu.VMEM((_NCHUNK, _ROWS, cols), x.dtype),
            pltpu.SemaphoreType.DMA((_NCHUNK,)),
            pltpu.SemaphoreType.DMA((_NCHUNK,)),
        ],
    )(x)
